# Initial kernel scaffold; baseline (speedup 1.0000x reference)
import functools

import jax
import jax.numpy as jnp
from jax import lax
from jax.experimental import pallas as pl
from jax.experimental.pallas import tpu as pltpu
from jax.experimental.pallas import tpu_sc as plsc

N = 10000
NP = 10240
E = 320000
D = 128
OUT = 2
NC = 2
NS = 16
NW = NC * NS
EW = E // NW
CH = 80
NCHUNK = EW // CH
RPT = NP // NS

_mesh = plsc.VectorSubcoreMesh(
    core_axis_name="c", subcore_axis_name="s", num_cores=NC, num_subcores=NS
)


@functools.partial(
    pl.kernel,
    out_type=jax.ShapeDtypeStruct((NC * NP,), jnp.float32),
    mesh=_mesh,
    scratch_types=[
        pltpu.VMEM((NCHUNK, CH), jnp.int32),
        pltpu.VMEM((CH,), jnp.float32),
        pltpu.VMEM_SHARED((NP,), jnp.float32),
    ],
)
def _deg_kernel(dst3, znode, out, dst_v, ones_v, acc_sh):
    c = lax.axis_index("c")
    s = lax.axis_index("s")
    wid = c * NS + s
    pltpu.sync_copy(znode.at[pl.ds(s * RPT, RPT)], acc_sh.at[pl.ds(s * RPT, RPT)])
    pltpu.sync_copy(dst3.at[wid], dst_v)
    for i in range(CH // 16):
        ones_v[pl.ds(i * 16, 16)] = jnp.ones((16,), jnp.float32)
    plsc.subcore_barrier()

    def body(j, carry):
        pltpu.sync_copy(ones_v, acc_sh.at[dst_v.at[j]], add=True)
        return carry

    lax.fori_loop(0, NCHUNK, body, 0)
    plsc.subcore_barrier()
    pltpu.sync_copy(acc_sh.at[pl.ds(s * RPT, RPT)], out.at[pl.ds(c * NP + s * RPT, RPT)])


@functools.partial(
    pl.kernel,
    out_type=jax.ShapeDtypeStruct((NC * NP, D), jnp.float32),
    mesh=_mesh,
    scratch_types=[
        pltpu.VMEM((NCHUNK, CH), jnp.int32),
        pltpu.VMEM((NCHUNK, CH), jnp.int32),
        pltpu.VMEM((CH, D), jnp.float32),
        pltpu.VMEM_SHARED((NP, D), jnp.float32),
        pltpu.SemaphoreType.DMA,
    ],
)
def _agg_kernel(table, src3, dst3, zfeat, out, src_v, dst_v, rows_v, acc_sh, sem):
    c = lax.axis_index("c")
    s = lax.axis_index("s")
    wid = c * NS + s
    pltpu.sync_copy(zfeat.at[pl.ds(s * RPT, RPT)], acc_sh.at[pl.ds(s * RPT, RPT)])
    pltpu.sync_copy(src3.at[wid], src_v)
    pltpu.sync_copy(dst3.at[wid], dst_v)
    plsc.subcore_barrier()

    def body(j, carry):
        pltpu.async_copy(table.at[src_v.at[j]], rows_v, sem).wait()
        pltpu.sync_copy(rows_v, acc_sh.at[dst_v.at[j]], add=True)
        return carry

    lax.fori_loop(0, NCHUNK, body, 0)
    plsc.subcore_barrier()
    pltpu.sync_copy(
        acc_sh.at[pl.ds(s * RPT, RPT)], out.at[pl.ds(c * NP + s * RPT, RPT)]
    )


def _tc1_body(x_ref, w_ref, degb_ref, xs_ref, dinv_ref):
    dinv = lax.rsqrt(degb_ref[...])
    g = jnp.dot(x_ref[...], w_ref[...], preferred_element_type=jnp.float32)
    xs_ref[...] = g * dinv
    dinv_ref[...] = dinv


_tc1 = pl.pallas_call(
    _tc1_body,
    grid=(NP // 1024,),
    in_specs=[
        pl.BlockSpec((1024, D), lambda i: (i, 0)),
        pl.BlockSpec((D, D), lambda i: (0, 0)),
        pl.BlockSpec((1024, D), lambda i: (i, 0)),
    ],
    out_specs=[pl.BlockSpec((1024, D), lambda i: (i, 0))] * 2,
    out_shape=[jax.ShapeDtypeStruct((NP, D), jnp.float32)] * 2,
)


def _tc2_body(sa_ref, sb_ref, xs_ref, dinv_ref, b_ref, w_ref, o_ref):
    dinv = dinv_ref[...]
    h = jnp.maximum(dinv * (sa_ref[...] + sb_ref[...] + xs_ref[...]) + b_ref[...], 0.0)
    o_ref[...] = jnp.dot(h, w_ref[...], preferred_element_type=jnp.float32) * dinv


_tc2 = pl.pallas_call(
    _tc2_body,
    grid=(NP // 1024,),
    in_specs=[
        pl.BlockSpec((1024, D), lambda i: (i, 0)),
        pl.BlockSpec((1024, D), lambda i: (i, 0)),
        pl.BlockSpec((1024, D), lambda i: (i, 0)),
        pl.BlockSpec((1024, D), lambda i: (i, 0)),
        pl.BlockSpec((1, D), lambda i: (0, 0)),
        pl.BlockSpec((D, D), lambda i: (0, 0)),
    ],
    out_specs=pl.BlockSpec((1024, D), lambda i: (i, 0)),
    out_shape=jax.ShapeDtypeStruct((NP, D), jnp.float32),
)


def _tc3_body(sa_ref, sb_ref, xs_ref, dinv_ref, b_ref, w_ref, bl_ref, o_ref):
    dinv = dinv_ref[...]
    h = jnp.maximum(dinv * (sa_ref[...] + sb_ref[...] + xs_ref[...]) + b_ref[...], 0.0)
    o_ref[...] = jnp.dot(h, w_ref[...], preferred_element_type=jnp.float32) + bl_ref[...]


_tc3 = pl.pallas_call(
    _tc3_body,
    grid=(NP // 1024,),
    in_specs=[
        pl.BlockSpec((1024, D), lambda i: (i, 0)),
        pl.BlockSpec((1024, D), lambda i: (i, 0)),
        pl.BlockSpec((1024, D), lambda i: (i, 0)),
        pl.BlockSpec((1024, D), lambda i: (i, 0)),
        pl.BlockSpec((1, D), lambda i: (0, 0)),
        pl.BlockSpec((D, D), lambda i: (0, 0)),
        pl.BlockSpec((1, D), lambda i: (0, 0)),
    ],
    out_specs=pl.BlockSpec((1024, D), lambda i: (i, 0)),
    out_shape=jax.ShapeDtypeStruct((NP, D), jnp.float32),
)


@jax.jit
def kernel(x, edge_index, W1, b1, W2, b2, Wlin, blin):
    src3 = edge_index[0].reshape(NW, NCHUNK, CH)
    dst3 = edge_index[1].reshape(NW, NCHUNK, CH)
    znode = jnp.zeros((NP,), jnp.float32)
    zfeat = jnp.zeros((NP, D), jnp.float32)

    degp = _deg_kernel(dst3, znode)
    deg = degp[:NP] + degp[NP:] + 1.0
    deg_b = jnp.broadcast_to(deg[:, None], (NP, D))

    x_pad = jnp.pad(x, ((0, NP - N), (0, 0)))
    xs1, dinv_b = _tc1(x_pad, W1, deg_b)

    s1 = _agg_kernel(xs1, src3, dst3, zfeat)
    xs2 = _tc2(s1[:NP], s1[NP:], xs1, dinv_b, b1.reshape(1, D), W2)

    s2 = _agg_kernel(xs2, src3, dst3, zfeat)
    Wlin_pad = jnp.pad(Wlin, ((0, 0), (0, D - OUT)))
    blin_pad = jnp.pad(blin, (0, D - OUT)).reshape(1, D)
    out = _tc3(s2[:NP], s2[NP:], xs2, dinv_b, b2.reshape(1, D), Wlin_pad, blin_pad)
    return out[:N, :OUT]


# speedup vs baseline: 16.9834x; 16.9834x over previous
import functools

import jax
import jax.numpy as jnp
from jax import lax
from jax.experimental import pallas as pl
from jax.experimental.pallas import tpu as pltpu
from jax.experimental.pallas import tpu_sc as plsc

N = 10000
NP = 10240
E = 320000
D = 128
OUT = 2
NC = 2
NS = 16
NW = NC * NS
EW = E // NW
CH = 80
NCHUNK = EW // CH
RPT = NP // NS

_mesh = plsc.VectorSubcoreMesh(
    core_axis_name="c", subcore_axis_name="s", num_cores=NC, num_subcores=NS
)


DL = 128


@functools.partial(
    pl.kernel,
    out_type=jax.ShapeDtypeStruct((NC * NP, DL), jnp.float32),
    mesh=_mesh,
    scratch_types=[
        pltpu.VMEM((NCHUNK, CH), jnp.int32),
        pltpu.VMEM((CH, DL), jnp.float32),
        pltpu.VMEM_SHARED((NP, DL), jnp.float32),
    ],
)
def _deg_kernel(dst3, znode, ones_h, out, dst_v, ones_v, acc_sh):
    c = lax.axis_index("c")
    s = lax.axis_index("s")
    wid = c * NS + s
    pltpu.sync_copy(znode.at[pl.ds(s * RPT, RPT)], acc_sh.at[pl.ds(s * RPT, RPT)])
    pltpu.sync_copy(dst3.at[wid], dst_v)
    pltpu.sync_copy(ones_h, ones_v)
    plsc.subcore_barrier()

    def body(j, carry):
        pltpu.sync_copy(ones_v, acc_sh.at[dst_v.at[j]], add=True)
        return carry

    lax.fori_loop(0, NCHUNK, body, 0)
    plsc.subcore_barrier()
    pltpu.sync_copy(acc_sh.at[pl.ds(s * RPT, RPT)], out.at[pl.ds(c * NP + s * RPT, RPT)])


@functools.partial(
    pl.kernel,
    out_type=jax.ShapeDtypeStruct((NC * NP, D), jnp.float32),
    mesh=_mesh,
    scratch_types=[
        pltpu.VMEM((NCHUNK, CH), jnp.int32),
        pltpu.VMEM((NCHUNK, CH), jnp.int32),
        pltpu.VMEM((CH, D), jnp.float32),
        pltpu.VMEM_SHARED((NP, D), jnp.float32),
        pltpu.SemaphoreType.DMA,
    ],
)
def _agg_kernel(table, src3, dst3, zfeat, out, src_v, dst_v, rows_v, acc_sh, sem):
    c = lax.axis_index("c")
    s = lax.axis_index("s")
    wid = c * NS + s
    pltpu.sync_copy(zfeat.at[pl.ds(s * RPT, RPT)], acc_sh.at[pl.ds(s * RPT, RPT)])
    pltpu.sync_copy(src3.at[wid], src_v)
    pltpu.sync_copy(dst3.at[wid], dst_v)
    plsc.subcore_barrier()

    def body(j, carry):
        pltpu.async_copy(table.at[src_v.at[j]], rows_v, sem).wait()
        pltpu.sync_copy(rows_v, acc_sh.at[dst_v.at[j]], add=True)
        return carry

    lax.fori_loop(0, NCHUNK, body, 0)
    plsc.subcore_barrier()
    pltpu.sync_copy(
        acc_sh.at[pl.ds(s * RPT, RPT)], out.at[pl.ds(c * NP + s * RPT, RPT)]
    )


def _tc1_body(x_ref, w_ref, degb_ref, xs_ref, dinv_ref):
    dinv = lax.rsqrt(degb_ref[...])
    g = jnp.dot(x_ref[...], w_ref[...], preferred_element_type=jnp.float32)
    xs_ref[...] = g * dinv
    dinv_ref[...] = dinv


_tc1 = pl.pallas_call(
    _tc1_body,
    grid=(NP // 1024,),
    in_specs=[
        pl.BlockSpec((1024, D), lambda i: (i, 0)),
        pl.BlockSpec((D, D), lambda i: (0, 0)),
        pl.BlockSpec((1024, D), lambda i: (i, 0)),
    ],
    out_specs=[pl.BlockSpec((1024, D), lambda i: (i, 0))] * 2,
    out_shape=[jax.ShapeDtypeStruct((NP, D), jnp.float32)] * 2,
)


def _tc2_body(sa_ref, sb_ref, xs_ref, dinv_ref, b_ref, w_ref, o_ref):
    dinv = dinv_ref[...]
    h = jnp.maximum(dinv * (sa_ref[...] + sb_ref[...] + xs_ref[...]) + b_ref[...], 0.0)
    o_ref[...] = jnp.dot(h, w_ref[...], preferred_element_type=jnp.float32) * dinv


_tc2 = pl.pallas_call(
    _tc2_body,
    grid=(NP // 1024,),
    in_specs=[
        pl.BlockSpec((1024, D), lambda i: (i, 0)),
        pl.BlockSpec((1024, D), lambda i: (i, 0)),
        pl.BlockSpec((1024, D), lambda i: (i, 0)),
        pl.BlockSpec((1024, D), lambda i: (i, 0)),
        pl.BlockSpec((1, D), lambda i: (0, 0)),
        pl.BlockSpec((D, D), lambda i: (0, 0)),
    ],
    out_specs=pl.BlockSpec((1024, D), lambda i: (i, 0)),
    out_shape=jax.ShapeDtypeStruct((NP, D), jnp.float32),
)


def _tc3_body(sa_ref, sb_ref, xs_ref, dinv_ref, b_ref, w_ref, bl_ref, o_ref):
    dinv = dinv_ref[...]
    h = jnp.maximum(dinv * (sa_ref[...] + sb_ref[...] + xs_ref[...]) + b_ref[...], 0.0)
    o_ref[...] = jnp.dot(h, w_ref[...], preferred_element_type=jnp.float32) + bl_ref[...]


_tc3 = pl.pallas_call(
    _tc3_body,
    grid=(NP // 1024,),
    in_specs=[
        pl.BlockSpec((1024, D), lambda i: (i, 0)),
        pl.BlockSpec((1024, D), lambda i: (i, 0)),
        pl.BlockSpec((1024, D), lambda i: (i, 0)),
        pl.BlockSpec((1024, D), lambda i: (i, 0)),
        pl.BlockSpec((1, D), lambda i: (0, 0)),
        pl.BlockSpec((D, D), lambda i: (0, 0)),
        pl.BlockSpec((1, D), lambda i: (0, 0)),
    ],
    out_specs=pl.BlockSpec((1024, D), lambda i: (i, 0)),
    out_shape=jax.ShapeDtypeStruct((NP, D), jnp.float32),
)


@jax.jit
def kernel(x, edge_index, W1, b1, W2, b2, Wlin, blin):
    src3 = edge_index[0].reshape(NW, NCHUNK, CH)
    dst3 = edge_index[1].reshape(NW, NCHUNK, CH)
    zfeat = jnp.zeros((NP, D), jnp.float32)
    ones_h = jnp.ones((CH, DL), jnp.float32)

    degp = _deg_kernel(dst3, zfeat, ones_h)
    deg = degp[:NP, 0] + degp[NP:, 0] + 1.0
    deg_b = jnp.broadcast_to(deg[:, None], (NP, D))

    x_pad = jnp.pad(x, ((0, NP - N), (0, 0)))
    xs1, dinv_b = _tc1(x_pad, W1, deg_b)

    s1 = _agg_kernel(xs1, src3, dst3, zfeat)
    xs2 = _tc2(s1[:NP], s1[NP:], xs1, dinv_b, b1.reshape(1, D), W2)

    s2 = _agg_kernel(xs2, src3, dst3, zfeat)
    Wlin_pad = jnp.pad(Wlin, ((0, 0), (0, D - OUT)))
    blin_pad = jnp.pad(blin, (0, D - OUT)).reshape(1, D)
    out = _tc3(s2[:NP], s2[NP:], xs2, dinv_b, b2.reshape(1, D), Wlin_pad, blin_pad)
    return out[:N, :OUT]


# speedup vs baseline: 18.0472x; 1.0626x over previous
import functools

import jax
import jax.numpy as jnp
from jax import lax
from jax.experimental import pallas as pl
from jax.experimental.pallas import tpu as pltpu
from jax.experimental.pallas import tpu_sc as plsc

N = 10000
NP = 10240
E = 320000
D = 128
OUT = 2
NC = 2
NS = 16
NW = NC * NS
EW = E // NW
CH = 100
NCHUNK = EW // CH
RPT = NP // NS

_mesh = plsc.VectorSubcoreMesh(
    core_axis_name="c", subcore_axis_name="s", num_cores=NC, num_subcores=NS
)


@functools.partial(
    pl.kernel,
    out_type=jax.ShapeDtypeStruct((NC * NP, D), jnp.float32),
    mesh=_mesh,
    scratch_types=[
        pltpu.VMEM((NCHUNK, CH), jnp.int32),
        pltpu.VMEM((CH, D), jnp.float32),
        pltpu.VMEM_SHARED((NP, D), jnp.float32),
    ],
)
def _deg_kernel(dst3, zfeat, ones_h, out, dst_v, ones_v, acc_sh):
    c = lax.axis_index("c")
    s = lax.axis_index("s")
    wid = c * NS + s
    pltpu.sync_copy(zfeat.at[pl.ds(s * RPT, RPT)], acc_sh.at[pl.ds(s * RPT, RPT)])
    pltpu.sync_copy(dst3.at[wid], dst_v)
    pltpu.sync_copy(ones_h, ones_v)
    plsc.subcore_barrier()

    def body(j, carry):
        pltpu.sync_copy(ones_v, acc_sh.at[dst_v.at[j]], add=True)
        return carry

    lax.fori_loop(0, NCHUNK, body, 0)
    plsc.subcore_barrier()
    pltpu.sync_copy(acc_sh.at[pl.ds(s * RPT, RPT)], out.at[pl.ds(c * NP + s * RPT, RPT)])


@functools.partial(
    pl.kernel,
    out_type=jax.ShapeDtypeStruct((NC * NP, D), jnp.float32),
    mesh=_mesh,
    scratch_types=[
        pltpu.VMEM((NCHUNK, CH), jnp.int32),
        pltpu.VMEM((NCHUNK, CH), jnp.int32),
        pltpu.VMEM((CH, D), jnp.float32),
        pltpu.VMEM_SHARED((NP, D), jnp.float32),
        pltpu.SemaphoreType.DMA,
    ],
)
def _agg_kernel(table, src3, dst3, zfeat, out, src_v, dst_v, rows_v, acc_sh, sem):
    c = lax.axis_index("c")
    s = lax.axis_index("s")
    wid = c * NS + s
    pltpu.sync_copy(zfeat.at[pl.ds(s * RPT, RPT)], acc_sh.at[pl.ds(s * RPT, RPT)])
    pltpu.sync_copy(src3.at[wid], src_v)
    pltpu.sync_copy(dst3.at[wid], dst_v)
    plsc.subcore_barrier()

    def body(j, carry):
        pltpu.async_copy(table.at[src_v.at[j]], rows_v, sem).wait()
        pltpu.sync_copy(rows_v, acc_sh.at[dst_v.at[j]], add=True)
        return carry

    lax.fori_loop(0, NCHUNK, body, 0)
    plsc.subcore_barrier()
    pltpu.sync_copy(
        acc_sh.at[pl.ds(s * RPT, RPT)], out.at[pl.ds(c * NP + s * RPT, RPT)]
    )


def _tc1_body(x_ref, w_ref, degb_ref, xs_ref, dinv_ref):
    dinv = lax.rsqrt(degb_ref[...])
    g = jnp.dot(x_ref[...], w_ref[...], preferred_element_type=jnp.float32)
    xs_ref[...] = g * dinv
    dinv_ref[...] = dinv


_tc1 = pl.pallas_call(
    _tc1_body,
    grid=(NP // 1024,),
    in_specs=[
        pl.BlockSpec((1024, D), lambda i: (i, 0)),
        pl.BlockSpec((D, D), lambda i: (0, 0)),
        pl.BlockSpec((1024, D), lambda i: (i, 0)),
    ],
    out_specs=[pl.BlockSpec((1024, D), lambda i: (i, 0))] * 2,
    out_shape=[jax.ShapeDtypeStruct((NP, D), jnp.float32)] * 2,
)


def _tc2_body(sa_ref, sb_ref, xs_ref, dinv_ref, b_ref, w_ref, o_ref):
    dinv = dinv_ref[...]
    h = jnp.maximum(dinv * (sa_ref[...] + sb_ref[...] + xs_ref[...]) + b_ref[...], 0.0)
    o_ref[...] = jnp.dot(h, w_ref[...], preferred_element_type=jnp.float32) * dinv


_tc2 = pl.pallas_call(
    _tc2_body,
    grid=(NP // 1024,),
    in_specs=[
        pl.BlockSpec((1024, D), lambda i: (i, 0)),
        pl.BlockSpec((1024, D), lambda i: (i, 0)),
        pl.BlockSpec((1024, D), lambda i: (i, 0)),
        pl.BlockSpec((1024, D), lambda i: (i, 0)),
        pl.BlockSpec((1, D), lambda i: (0, 0)),
        pl.BlockSpec((D, D), lambda i: (0, 0)),
    ],
    out_specs=pl.BlockSpec((1024, D), lambda i: (i, 0)),
    out_shape=jax.ShapeDtypeStruct((NP, D), jnp.float32),
)


def _tc3_body(sa_ref, sb_ref, xs_ref, dinv_ref, b_ref, w_ref, bl_ref, o_ref):
    dinv = dinv_ref[...]
    h = jnp.maximum(dinv * (sa_ref[...] + sb_ref[...] + xs_ref[...]) + b_ref[...], 0.0)
    o_ref[...] = jnp.dot(h, w_ref[...], preferred_element_type=jnp.float32) + bl_ref[...]


_tc3 = pl.pallas_call(
    _tc3_body,
    grid=(NP // 1024,),
    in_specs=[
        pl.BlockSpec((1024, D), lambda i: (i, 0)),
        pl.BlockSpec((1024, D), lambda i: (i, 0)),
        pl.BlockSpec((1024, D), lambda i: (i, 0)),
        pl.BlockSpec((1024, D), lambda i: (i, 0)),
        pl.BlockSpec((1, D), lambda i: (0, 0)),
        pl.BlockSpec((D, D), lambda i: (0, 0)),
        pl.BlockSpec((1, D), lambda i: (0, 0)),
    ],
    out_specs=pl.BlockSpec((1024, D), lambda i: (i, 0)),
    out_shape=jax.ShapeDtypeStruct((NP, D), jnp.float32),
)


@jax.jit
def kernel(x, edge_index, W1, b1, W2, b2, Wlin, blin):
    src3 = edge_index[0].reshape(NW, NCHUNK, CH)
    dst3 = edge_index[1].reshape(NW, NCHUNK, CH)
    zfeat = jnp.zeros((NP, D), jnp.float32)
    ones_h = jnp.ones((CH, D), jnp.float32)

    degp = _deg_kernel(dst3, zfeat, ones_h)
    deg = degp[:NP, 0] + degp[NP:, 0] + 1.0
    deg_b = jnp.broadcast_to(deg[:, None], (NP, D))

    x_pad = jnp.pad(x, ((0, NP - N), (0, 0)))
    xs1, dinv_b = _tc1(x_pad, W1, deg_b)

    s1 = _agg_kernel(xs1, src3, dst3, zfeat)
    xs2 = _tc2(s1[:NP], s1[NP:], xs1, dinv_b, b1.reshape(1, D), W2)

    s2 = _agg_kernel(xs2, src3, dst3, zfeat)
    Wlin_pad = jnp.pad(Wlin, ((0, 0), (0, D - OUT)))
    blin_pad = jnp.pad(blin, (0, D - OUT)).reshape(1, D)
    out = _tc3(s2[:NP], s2[NP:], xs2, dinv_b, b2.reshape(1, D), Wlin_pad, blin_pad)
    return out[:N, :OUT]


# speedup vs baseline: 18.3905x; 1.0190x over previous
import functools

import jax
import jax.numpy as jnp
from jax import lax
from jax.experimental import pallas as pl
from jax.experimental.pallas import tpu as pltpu
from jax.experimental.pallas import tpu_sc as plsc

N = 10000
NP = 10240
E = 320000
D = 128
OUT = 2
NC = 2
NS = 16
NW = NC * NS
EW = E // NW
CH = 100
NCHUNK = EW // CH
RPT = NP // NS

_mesh = plsc.VectorSubcoreMesh(
    core_axis_name="c", subcore_axis_name="s", num_cores=NC, num_subcores=NS
)


@functools.partial(
    pl.kernel,
    out_type=jax.ShapeDtypeStruct((NC * NP, D), jnp.float32),
    mesh=_mesh,
    scratch_types=[
        pltpu.VMEM((NCHUNK, CH), jnp.int32),
        pltpu.VMEM((CH, D), jnp.float32),
        pltpu.VMEM_SHARED((NP, D), jnp.float32),
    ],
)
def _deg_kernel(dst3, zfeat, ones_h, out, dst_v, ones_v, acc_sh):
    c = lax.axis_index("c")
    s = lax.axis_index("s")
    wid = c * NS + s
    pltpu.sync_copy(zfeat.at[pl.ds(s * RPT, RPT)], acc_sh.at[pl.ds(s * RPT, RPT)])
    pltpu.sync_copy(dst3.at[wid], dst_v)
    pltpu.sync_copy(ones_h, ones_v)
    plsc.subcore_barrier()

    def body(j, carry):
        pltpu.sync_copy(ones_v, acc_sh.at[dst_v.at[j]], add=True)
        return carry

    lax.fori_loop(0, NCHUNK, body, 0)
    plsc.subcore_barrier()
    pltpu.sync_copy(acc_sh.at[pl.ds(s * RPT, RPT)], out.at[pl.ds(c * NP + s * RPT, RPT)])


@functools.partial(
    pl.kernel,
    out_type=jax.ShapeDtypeStruct((NC * NP, D), jnp.float32),
    mesh=_mesh,
    scratch_types=[
        pltpu.VMEM((NCHUNK, CH), jnp.int32),
        pltpu.VMEM((NCHUNK, CH), jnp.int32),
        pltpu.VMEM((CH, D), jnp.float32),
        pltpu.VMEM_SHARED((NP, D), jnp.float32),
        pltpu.SemaphoreType.DMA,
    ],
)
def _agg_kernel(table, idx4, zfeat, out, src_v, dst_v, rows_v, acc_sh, sem):
    c = lax.axis_index("c")
    s = lax.axis_index("s")
    wid = c * NS + s
    pltpu.sync_copy(zfeat.at[pl.ds(s * RPT, RPT)], acc_sh.at[pl.ds(s * RPT, RPT)])
    pltpu.sync_copy(idx4.at[0, wid], src_v)
    pltpu.sync_copy(idx4.at[1, wid], dst_v)
    plsc.subcore_barrier()

    def body(j, carry):
        pltpu.async_copy(table.at[src_v.at[j]], rows_v, sem).wait()
        pltpu.sync_copy(rows_v, acc_sh.at[dst_v.at[j]], add=True)
        return carry

    lax.fori_loop(0, NCHUNK, body, 0)
    plsc.subcore_barrier()
    pltpu.sync_copy(
        acc_sh.at[pl.ds(s * RPT, RPT)], out.at[pl.ds(c * NP + s * RPT, RPT)]
    )


def _tc1_body(x_ref, w_ref, degb_ref, xs_ref, dinv_ref):
    dinv = lax.rsqrt(degb_ref[...])
    g = jnp.dot(x_ref[...], w_ref[...], preferred_element_type=jnp.float32)
    xs_ref[...] = g * dinv
    dinv_ref[...] = dinv


_tc1 = pl.pallas_call(
    _tc1_body,
    grid=(NP // 1024,),
    in_specs=[
        pl.BlockSpec((1024, D), lambda i: (i, 0)),
        pl.BlockSpec((D, D), lambda i: (0, 0)),
        pl.BlockSpec((1024, D), lambda i: (i, 0)),
    ],
    out_specs=[pl.BlockSpec((1024, D), lambda i: (i, 0))] * 2,
    out_shape=[jax.ShapeDtypeStruct((NP, D), jnp.float32)] * 2,
)


def _tc2_body(sa_ref, sb_ref, xs_ref, dinv_ref, b_ref, w_ref, o_ref):
    dinv = dinv_ref[...]
    h = jnp.maximum(dinv * (sa_ref[...] + sb_ref[...] + xs_ref[...]) + b_ref[...], 0.0)
    o_ref[...] = jnp.dot(h, w_ref[...], preferred_element_type=jnp.float32) * dinv


_tc2 = pl.pallas_call(
    _tc2_body,
    grid=(NP // 1024,),
    in_specs=[
        pl.BlockSpec((1024, D), lambda i: (i, 0)),
        pl.BlockSpec((1024, D), lambda i: (i, 0)),
        pl.BlockSpec((1024, D), lambda i: (i, 0)),
        pl.BlockSpec((1024, D), lambda i: (i, 0)),
        pl.BlockSpec((1, D), lambda i: (0, 0)),
        pl.BlockSpec((D, D), lambda i: (0, 0)),
    ],
    out_specs=pl.BlockSpec((1024, D), lambda i: (i, 0)),
    out_shape=jax.ShapeDtypeStruct((NP, D), jnp.float32),
)


def _tc3_body(sa_ref, sb_ref, xs_ref, dinv_ref, b_ref, w_ref, bl_ref, o_ref):
    dinv = dinv_ref[...]
    h = jnp.maximum(dinv * (sa_ref[...] + sb_ref[...] + xs_ref[...]) + b_ref[...], 0.0)
    o_ref[...] = jnp.dot(h, w_ref[...], preferred_element_type=jnp.float32) + bl_ref[...]


_tc3 = pl.pallas_call(
    _tc3_body,
    grid=(NP // 1024,),
    in_specs=[
        pl.BlockSpec((1024, D), lambda i: (i, 0)),
        pl.BlockSpec((1024, D), lambda i: (i, 0)),
        pl.BlockSpec((1024, D), lambda i: (i, 0)),
        pl.BlockSpec((1024, D), lambda i: (i, 0)),
        pl.BlockSpec((1, D), lambda i: (0, 0)),
        pl.BlockSpec((D, D), lambda i: (0, 0)),
        pl.BlockSpec((1, D), lambda i: (0, 0)),
    ],
    out_specs=pl.BlockSpec((1024, D), lambda i: (i, 0)),
    out_shape=jax.ShapeDtypeStruct((NP, D), jnp.float32),
)


@jax.jit
def kernel(x, edge_index, W1, b1, W2, b2, Wlin, blin):
    idx4 = edge_index.reshape(2, NW, NCHUNK, CH)
    dst3 = idx4[1]
    zfeat = jnp.zeros((NP, D), jnp.float32)
    ones_h = jnp.ones((CH, D), jnp.float32)

    degp = _deg_kernel(dst3, zfeat, ones_h)
    deg = degp[:NP, 0] + degp[NP:, 0] + 1.0
    deg_b = jnp.broadcast_to(deg[:, None], (NP, D))

    x_pad = jnp.pad(x, ((0, NP - N), (0, 0)))
    xs1, dinv_b = _tc1(x_pad, W1, deg_b)

    s1 = _agg_kernel(xs1, idx4, zfeat)
    xs2 = _tc2(s1[:NP], s1[NP:], xs1, dinv_b, b1.reshape(1, D), W2)

    s2 = _agg_kernel(xs2, idx4, zfeat)
    Wlin_pad = jnp.pad(Wlin, ((0, 0), (0, D - OUT)))
    blin_pad = jnp.pad(blin, (0, D - OUT)).reshape(1, D)
    out = _tc3(s2[:NP], s2[NP:], xs2, dinv_b, b2.reshape(1, D), Wlin_pad, blin_pad)
    return out[:N, :OUT]
